# Initial kernel scaffold; baseline (speedup 1.0000x reference)
import jax
import jax.numpy as jnp
from jax.experimental import pallas as pl


def kernel(x, w1, b1, w2, b2, w3, b3, w4, b4, w5, b5, w6, b6, w7, b7):
    raise NotImplementedError("write your pallas kernel here")


# speedup vs baseline: 1.7069x; 1.7069x over previous
import functools

import numpy as np
import jax
import jax.numpy as jnp
from jax.experimental import pallas as pl
from jax.experimental.pallas import tpu as pltpu

_ITERS = 8
_CH = 3


def _col_masks(H, W):
    xx = np.tile(np.arange(W), H)
    m = np.zeros((2, 1, H * W), np.float32)
    m[0, 0] = (xx > 0).astype(np.float32)
    m[1, 0] = (xx < W - 1).astype(np.float32)
    return jnp.asarray(m)


def _wd(w):
    cout, cin = w.shape[0], w.shape[1]
    return jnp.transpose(w, (3, 0, 2, 1)).reshape(3 * cout, 3 * cin).astype(jnp.bfloat16)


def _dce_kernel(x_ref, masks_ref,
                wd1, b1, wd2, b2, wd3, b3, wd4, b4,
                wd5, b5, wd6, b6, wd7, b7,
                xe_ref, xr_ref, *, H, W, iters):
    HW = H * W
    ml = masks_ref[0]
    mr = masks_ref[1]

    def packv(xb):
        c = xb.shape[0]
        z = jnp.zeros((c, W), xb.dtype)
        up = jnp.concatenate([z, xb[:, :HW - W]], axis=1)
        dn = jnp.concatenate([xb[:, W:], z], axis=1)
        return jnp.concatenate([up, xb, dn], axis=0)

    def conv(packed, wd_ref, b_ref, act):
        q = jnp.dot(wd_ref[...], packed, preferred_element_type=jnp.float32)
        cout = q.shape[0] // 3
        qm, q0, qp = q[:cout], q[cout:2 * cout], q[2 * cout:]
        y = (q0 + ml * pltpu.roll(qm, 1, 1)
                + mr * pltpu.roll(qp, HW - 1, 1)) + b_ref[...]
        if act == "relu":
            y = jnp.maximum(y, 0.0)
        else:
            y = jnp.tanh(y)
        return y

    x0 = x_ref[...]
    x1 = conv(packv(x0.astype(jnp.bfloat16)), wd1, b1, "relu")
    x2 = conv(packv(x1.astype(jnp.bfloat16)), wd2, b2, "relu")
    p2 = packv(x2.astype(jnp.bfloat16))
    x3 = conv(p2, wd3, b3, "relu")
    p3 = packv(x3.astype(jnp.bfloat16))
    x4 = conv(p3, wd4, b4, "relu")
    x5 = conv(jnp.concatenate([p3, packv(x4.astype(jnp.bfloat16))], axis=0),
              wd5, b5, "relu")
    x6 = conv(jnp.concatenate([p2, packv(x5.astype(jnp.bfloat16))], axis=0),
              wd6, b6, "relu")
    xr = conv(packv(x6.astype(jnp.bfloat16)), wd7, b7, "tanh")

    xe = x0
    for i in range(iters):
        ri = xr[i * _CH:(i + 1) * _CH]
        xe = xe + ri * (xe * xe - xe)
        xe = jnp.clip(xe, 0.0, 1.0)

    xe_ref[...] = xe
    xr_ref[...] = xr


def _const_spec(arr):
    zeros = (0,) * arr.ndim

    def index_map(n):
        return zeros

    return pl.BlockSpec(arr.shape, index_map)


def kernel(x, w1, b1, w2, b2, w3, b3, w4, b4, w5, b5, w6, b6, w7, b7):
    N, C, H, W = x.shape
    HW = H * W
    CR = _CH * _ITERS

    xf = x.reshape(N, C, HW).astype(jnp.float32)
    masks = _col_masks(H, W)

    h5 = w5.shape[1] // 2
    h6 = w6.shape[1] // 2
    wd5 = jnp.concatenate([_wd(w5[:, :h5]), _wd(w5[:, h5:])], axis=1)
    wd6 = jnp.concatenate([_wd(w6[:, :h6]), _wd(w6[:, h6:])], axis=1)

    def rb(b):
        return b.reshape(-1, 1).astype(jnp.float32)

    flat = [_wd(w1), rb(b1), _wd(w2), rb(b2), _wd(w3), rb(b3), _wd(w4), rb(b4),
            wd5, rb(b5), wd6, rb(b6), _wd(w7), rb(b7)]

    body = functools.partial(_dce_kernel, H=H, W=W, iters=_ITERS)

    in_specs = ([pl.BlockSpec((None, C, HW), lambda n: (n, 0, 0)),
                 _const_spec(masks)]
                + [_const_spec(p) for p in flat])

    xe, xr = pl.pallas_call(
        body,
        out_shape=(jax.ShapeDtypeStruct((N, C, HW), jnp.float32),
                   jax.ShapeDtypeStruct((N, CR, HW), jnp.float32)),
        grid_spec=pltpu.PrefetchScalarGridSpec(
            num_scalar_prefetch=0,
            grid=(N,),
            in_specs=in_specs,
            out_specs=(pl.BlockSpec((None, C, HW), lambda n: (n, 0, 0)),
                       pl.BlockSpec((None, CR, HW), lambda n: (n, 0, 0))),
        ),
        compiler_params=pltpu.CompilerParams(
            dimension_semantics=("parallel",)),
    )(xf, masks, *flat)

    x_enhanced = xe.reshape(N, C, H, W)
    x_r = xr.reshape(N, CR, H, W)
    r = tuple(x_r[:, i * _CH:(i + 1) * _CH] for i in range(_ITERS))
    return x_enhanced, r
